# baseline (speedup 1.0000x reference)
import functools

import jax
import jax.numpy as jnp
from jax import lax
from jax.experimental import pallas as pl
from jax.experimental.pallas import tpu as pltpu
from jax.experimental.pallas import tpu_sc as plsc

BATCH = 4096
SEQ = 200
D = 64
OUT_DIM = 1000
VOCAB_ROWS = 1000000

NC, NS = 2, 16
NW = NC * NS
ROWS_PER_W = BATCH // NW
CHUNK = 40
NCHUNK = SEQ // CHUNK

N_FULL_COLS = VOCAB_ROWS // 128
TAIL_V = N_FULL_COLS * 128
TAIL_N = VOCAB_ROWS - TAIL_V
U_ITERS = (N_FULL_COLS + NW - 1) // NW
CHUNK_WORDS = 128 * D


def _untile_body(tt_hbm, tail_hbm, out_hbm,
                 chunk0, chunk1, outv0, outv1, tail_v,
                 sem_i0, sem_i1, sem_o0, sem_o1):
    wid = lax.axis_index("s") * NC + lax.axis_index("c")

    def issue_in(c, buf, sem):
        pltpu.async_copy(tt_hbm.at[:, pl.ds(c * 128, 128)], buf, sem)

    def wait_in(buf, sem):
        pltpu.make_async_copy(tt_hbm.at[:, pl.ds(0, 128)], buf, sem).wait()

    iotas = [lax.iota(jnp.int32, 16) + 16 * k for k in range(4)]

    def transpose_chunk(buf, outv):
        def body(v, carry):
            vv = jnp.zeros((16,), jnp.int32) + v
            for k in range(4):
                g = plsc.load_gather(buf, [iotas[k], vv])
                outv[pl.ds(v * D + 16 * k, 16)] = g
            return carry
        lax.fori_loop(0, 128, body, 0, unroll=4)

    @pl.when(wid == 0)
    def _():
        pltpu.sync_copy(tail_hbm, tail_v)
        pltpu.sync_copy(tail_v, out_hbm.at[pl.ds(TAIL_V * D, TAIL_N * D)])

    c0 = wid
    @pl.when(c0 < N_FULL_COLS)
    def _():
        issue_in(c0, chunk0, sem_i0)

    bufs = ((chunk0, sem_i0, outv0, sem_o0), (chunk1, sem_i1, outv1, sem_o1))

    def outer(o, carry):
        for b in range(2):
            i = o * 2 + b
            c = i * NW + wid
            chunk, sem_i, outv, sem_o = bufs[b]
            nchunk, nsem_i, _, _ = bufs[1 - b]
            cn = c + NW

            @pl.when(cn < N_FULL_COLS)
            def _():
                issue_in(cn, nchunk, nsem_i)

            @pl.when(c < N_FULL_COLS)
            def _():
                wait_in(chunk, sem_i)

                @pl.when(i >= 2)
                def _():
                    pltpu.make_async_copy(
                        outv, out_hbm.at[pl.ds(0, CHUNK_WORDS)], sem_o).wait()

                transpose_chunk(chunk, outv)
                pltpu.async_copy(
                    outv, out_hbm.at[pl.ds(c * CHUNK_WORDS, CHUNK_WORDS)], sem_o)
        return carry

    lax.fori_loop(0, (U_ITERS + 1) // 2, outer, 0)

    pltpu.make_async_copy(outv0, out_hbm.at[pl.ds(0, CHUNK_WORDS)], sem_o0).wait()
    pltpu.make_async_copy(outv1, out_hbm.at[pl.ds(0, CHUNK_WORDS)], sem_o1).wait()


@functools.lru_cache(maxsize=None)
def _make_untile():
  return pl.kernel(
    _untile_body,
    out_type=jax.ShapeDtypeStruct((VOCAB_ROWS * D,), jnp.float32),
    mesh=plsc.VectorSubcoreMesh(core_axis_name="c", subcore_axis_name="s",
                                num_cores=NC, num_subcores=NS),
    scratch_types=[
        pltpu.VMEM((D, 128), jnp.float32),
        pltpu.VMEM((D, 128), jnp.float32),
        pltpu.VMEM((CHUNK_WORDS,), jnp.float32),
        pltpu.VMEM((CHUNK_WORDS,), jnp.float32),
        pltpu.VMEM((TAIL_N * D,), jnp.float32),
        pltpu.SemaphoreType.DMA,
        pltpu.SemaphoreType.DMA,
        pltpu.SemaphoreType.DMA,
        pltpu.SemaphoreType.DMA,
    ],
    compiler_params=pltpu.CompilerParams(needs_layout_passes=False),
  )


def _bag_body(idx_hbm, table_hbm, out_hbm, idx_v, rows0, rows1, out_v, sem0, sem1):
    wid = lax.axis_index("s") * NC + lax.axis_index("c")
    base = wid * ROWS_PER_W

    pltpu.sync_copy(idx_hbm.at[pl.ds(base, ROWS_PER_W), :], idx_v)

    def issue(r, buf, sem):
        for j in range(NCHUNK):
            pltpu.async_copy(
                table_hbm.at[idx_v.at[r, pl.ds(j * CHUNK, CHUNK)]],
                buf.at[pl.ds(j * CHUNK, CHUNK), :],
                sem,
            )

    def drain(buf, sem):
        for j in range(NCHUNK):
            pltpu.make_async_copy(
                table_hbm.at[idx_v.at[0, pl.ds(0, CHUNK)]],
                buf.at[pl.ds(j * CHUNK, CHUNK), :],
                sem,
            ).wait()

    def reduce_row(buf, r):
        def body(i, accs):
            return tuple(a + buf[i, pl.ds(d * 16, 16)] for d, a in enumerate(accs))
        accs = lax.fori_loop(0, SEQ, body,
                             tuple(jnp.zeros((16,), jnp.float32) for _ in range(4)),
                             unroll=8)
        for d in range(4):
            out_v[r, pl.ds(d * 16, 16)] = accs[d]

    issue(0, rows0, sem0)
    bufs = ((rows0, sem0), (rows1, sem1))

    def outer(o, carry):
        for b in range(2):
            r = o * 2 + b
            buf, sem = bufs[b]
            nbuf, nsem = bufs[1 - b]

            @pl.when(r + 1 < ROWS_PER_W)
            def _():
                issue(r + 1, nbuf, nsem)

            drain(buf, sem)
            reduce_row(buf, r)
        return carry

    lax.fori_loop(0, ROWS_PER_W // 2, outer, 0)
    pltpu.sync_copy(out_v, out_hbm.at[pl.ds(base, ROWS_PER_W), :])


@functools.lru_cache(maxsize=None)
def _make_bag_sum():
  return pl.kernel(
    _bag_body,
    out_type=jax.ShapeDtypeStruct((BATCH, D), jnp.float32),
    mesh=plsc.VectorSubcoreMesh(core_axis_name="c", subcore_axis_name="s",
                                num_cores=NC, num_subcores=NS),
    scratch_types=[
        pltpu.VMEM((ROWS_PER_W, SEQ), jnp.int32),
        pltpu.VMEM((SEQ, D), jnp.float32),
        pltpu.VMEM((SEQ, D), jnp.float32),
        pltpu.VMEM((ROWS_PER_W, D), jnp.float32),
        pltpu.SemaphoreType.DMA,
        pltpu.SemaphoreType.DMA,
    ],
    compiler_params=pltpu.CompilerParams(use_tc_tiling_on_sc=False),
  )


BT = 512


def _fc_body(emb_ref, w_ref, b_ref, out_ref):
    emb = emb_ref[...]
    norm = jnp.sqrt(jnp.sum(emb * emb, axis=1, keepdims=True))
    embn = emb / jnp.maximum(norm, 1e-12)
    out = lax.dot_general(embn, w_ref[...], (((1,), (1,)), ((), ())),
                          preferred_element_type=jnp.float32)
    out_ref[...] = out + b_ref[...]


def _fc(sums, W, b2d):
    return pl.pallas_call(
        _fc_body,
        grid=(BATCH // BT,),
        in_specs=[
            pl.BlockSpec((BT, D), lambda i: (i, 0)),
            pl.BlockSpec((OUT_DIM, D), lambda i: (0, 0)),
            pl.BlockSpec((1, OUT_DIM), lambda i: (0, 0)),
        ],
        out_specs=pl.BlockSpec((BT, OUT_DIM), lambda i: (i, 0)),
        out_shape=jax.ShapeDtypeStruct((BATCH, OUT_DIM), jnp.float32),
    )(sums, W, b2d)


def kernel(name_idxs, name_len, desc_idxs, desc_len, union_idxs, union_len, table, W, b):
    idx = union_idxs.astype(jnp.int32)
    tail = table[TAIL_V:, :].reshape(TAIL_N * D)
    table_lin = _make_untile()(table.T, tail).reshape(VOCAB_ROWS, D)
    sums = _make_bag_sum()(idx, table_lin)
    return _fc(sums, W, b.reshape(1, OUT_DIM))


# speedup vs baseline: 3.2010x; 3.2010x over previous
import functools

import jax
import jax.numpy as jnp
from jax import lax
from jax.experimental import pallas as pl
from jax.experimental.pallas import tpu as pltpu
from jax.experimental.pallas import tpu_sc as plsc

BATCH = 4096
SEQ = 200
D = 64
OUT_DIM = 1000
VOCAB_ROWS = 1000000

NC, NS = 2, 16
NW = NC * NS
ROWS_PER_W = BATCH // NW
CHUNK = 40
NCHUNK = SEQ // CHUNK

TC_COLS = 1280
N_RELAYOUT = 391
SPLIT = TC_COLS * N_RELAYOUT


def _relayout_body(lo_ref, hi_ref, out_ref):
    stacked = jnp.concatenate([lo_ref[...], hi_ref[...]], axis=0)
    row = jax.lax.broadcasted_iota(jnp.int32, (2 * D, 2 * D), 0)
    col = jax.lax.broadcasted_iota(jnp.int32, (2 * D, 2 * D), 1)
    ident = (row == col).astype(jnp.float32)
    out_ref[...] = jax.lax.dot_general(
        stacked, ident, (((0,), (0,)), ((), ())),
        preferred_element_type=jnp.float32)


def _relayout(tableT):
    return pl.pallas_call(
        _relayout_body,
        grid=(N_RELAYOUT,),
        in_specs=[
            pl.BlockSpec((D, TC_COLS), lambda i: (0, i)),
            pl.BlockSpec((D, TC_COLS), lambda i: (0, i + N_RELAYOUT)),
        ],
        out_specs=pl.BlockSpec((TC_COLS, 2 * D), lambda i: (i, 0)),
        out_shape=jax.ShapeDtypeStruct((SPLIT, 2 * D), jnp.float32),
    )(tableT, tableT)


def _bag_body(idx_hbm, table_hbm, out_hbm, idx_v, rows0, rows1, out_v, sem0, sem1):
    wid = lax.axis_index("s") * NC + lax.axis_index("c")
    base = wid * ROWS_PER_W

    pltpu.sync_copy(idx_hbm.at[pl.ds(base, ROWS_PER_W), :], idx_v)

    def issue(r, buf, sem):
        for j in range(NCHUNK):
            pltpu.async_copy(
                table_hbm.at[idx_v.at[r, pl.ds(j * CHUNK, CHUNK)]],
                buf.at[pl.ds(j * CHUNK, CHUNK), :],
                sem,
            )

    def drain(buf, sem):
        for j in range(NCHUNK):
            pltpu.make_async_copy(
                table_hbm.at[idx_v.at[0, pl.ds(0, CHUNK)]],
                buf.at[pl.ds(j * CHUNK, CHUNK), :],
                sem,
            ).wait()

    def reduce_row(buf, r):
        def body(i, accs):
            return tuple(a + buf[i, pl.ds(d * 16, 16)] for d, a in enumerate(accs))
        accs = lax.fori_loop(0, SEQ, body,
                             tuple(jnp.zeros((16,), jnp.float32) for _ in range(4)),
                             unroll=8)
        for d in range(4):
            out_v[r, pl.ds(d * 16, 16)] = accs[d]

    issue(0, rows0, sem0)
    bufs = ((rows0, sem0), (rows1, sem1))

    def outer(o, carry):
        for b in range(2):
            r = o * 2 + b
            buf, sem = bufs[b]
            nbuf, nsem = bufs[1 - b]

            @pl.when(r + 1 < ROWS_PER_W)
            def _():
                issue(r + 1, nbuf, nsem)

            drain(buf, sem)
            reduce_row(buf, r)
        return carry

    lax.fori_loop(0, ROWS_PER_W // 2, outer, 0)
    pltpu.sync_copy(out_v, out_hbm.at[pl.ds(base, ROWS_PER_W), :])


@functools.lru_cache(maxsize=None)
def _make_bag_sum():
  return pl.kernel(
    _bag_body,
    out_type=jax.ShapeDtypeStruct((BATCH, D), jnp.float32),
    mesh=plsc.VectorSubcoreMesh(core_axis_name="c", subcore_axis_name="s",
                                num_cores=NC, num_subcores=NS),
    scratch_types=[
        pltpu.VMEM((ROWS_PER_W, SEQ), jnp.int32),
        pltpu.VMEM((SEQ, D), jnp.float32),
        pltpu.VMEM((SEQ, D), jnp.float32),
        pltpu.VMEM((ROWS_PER_W, D), jnp.float32),
        pltpu.SemaphoreType.DMA,
        pltpu.SemaphoreType.DMA,
    ],
    compiler_params=pltpu.CompilerParams(use_tc_tiling_on_sc=False),
  )


BT = 512


def _fc_body(emb_ref, w_ref, b_ref, out_ref):
    emb = emb_ref[...]
    norm = jnp.sqrt(jnp.sum(emb * emb, axis=1, keepdims=True))
    embn = emb / jnp.maximum(norm, 1e-12)
    out = lax.dot_general(embn, w_ref[...], (((1,), (1,)), ((), ())),
                          preferred_element_type=jnp.float32)
    out_ref[...] = out + b_ref[...]


def _fc(sums, W, b2d):
    return pl.pallas_call(
        _fc_body,
        grid=(BATCH // BT,),
        in_specs=[
            pl.BlockSpec((BT, D), lambda i: (i, 0)),
            pl.BlockSpec((OUT_DIM, D), lambda i: (0, 0)),
            pl.BlockSpec((1, OUT_DIM), lambda i: (0, 0)),
        ],
        out_specs=pl.BlockSpec((BT, OUT_DIM), lambda i: (i, 0)),
        out_shape=jax.ShapeDtypeStruct((BATCH, OUT_DIM), jnp.float32),
    )(sums, W, b2d)


def kernel(name_idxs, name_len, desc_idxs, desc_len, union_idxs, union_len, table, W, b):
    idx = union_idxs.astype(jnp.int32)
    idx = jnp.where(idx < SPLIT, 2 * idx, 2 * (idx - SPLIT) + 1)
    table_lin = _relayout(table.T).reshape(2 * SPLIT, D)
    sums = _make_bag_sum()(idx, table_lin)
    return _fc(sums, W, b.reshape(1, OUT_DIM))


# speedup vs baseline: 4.6453x; 1.4512x over previous
import functools

import jax
import jax.numpy as jnp
from jax import lax
from jax.experimental import pallas as pl
from jax.experimental.pallas import tpu as pltpu
from jax.experimental.pallas import tpu_sc as plsc

BATCH = 4096
SEQ = 200
D = 64
OUT_DIM = 1000
VOCAB_ROWS = 1000000

NC, NS = 2, 16
NW = NC * NS
ROWS_PER_W = BATCH // NW
CHUNK = 40
NCHUNK = SEQ // CHUNK

TC_COLS = 4352
N_RELAYOUT = 115
SPLIT = TC_COLS * N_RELAYOUT


def _relayout_body(lo_ref, hi_ref, out_ref):
    stacked = jnp.concatenate([lo_ref[...], hi_ref[...]], axis=0)
    row = jax.lax.broadcasted_iota(jnp.int32, (2 * D, 2 * D), 0)
    col = jax.lax.broadcasted_iota(jnp.int32, (2 * D, 2 * D), 1)
    ident = (row == col).astype(jnp.float32)
    out_ref[...] = jax.lax.dot_general(
        stacked, ident, (((0,), (0,)), ((), ())),
        preferred_element_type=jnp.float32)


def _relayout(tableT):
    return pl.pallas_call(
        _relayout_body,
        grid=(N_RELAYOUT,),
        in_specs=[
            pl.BlockSpec((D, TC_COLS), lambda i: (0, i)),
            pl.BlockSpec((D, TC_COLS), lambda i: (0, i + N_RELAYOUT)),
        ],
        out_specs=pl.BlockSpec((TC_COLS, 2 * D), lambda i: (i, 0)),
        out_shape=jax.ShapeDtypeStruct((SPLIT, 2 * D), jnp.float32),
    )(tableT, tableT)


def _bag_body(idx_hbm, table_hbm, out_hbm, idx_v, rows0, rows1, out_v, sem0, sem1):
    wid = lax.axis_index("s") * NC + lax.axis_index("c")
    base = wid * ROWS_PER_W

    pltpu.sync_copy(idx_hbm.at[pl.ds(base, ROWS_PER_W), :], idx_v)

    def issue(r, buf, sem):
        for j in range(NCHUNK):
            pltpu.async_copy(
                table_hbm.at[idx_v.at[r, pl.ds(j * CHUNK, CHUNK)]],
                buf.at[pl.ds(j * CHUNK, CHUNK), :],
                sem,
            )

    def drain(buf, sem):
        for j in range(NCHUNK):
            pltpu.make_async_copy(
                table_hbm.at[idx_v.at[0, pl.ds(0, CHUNK)]],
                buf.at[pl.ds(j * CHUNK, CHUNK), :],
                sem,
            ).wait()

    def reduce_row(buf, r):
        def body(i, accs):
            return tuple(a + buf[i, pl.ds(d * 16, 16)] for d, a in enumerate(accs))
        accs = lax.fori_loop(0, SEQ, body,
                             tuple(jnp.zeros((16,), jnp.float32) for _ in range(4)),
                             unroll=8)
        for d in range(4):
            out_v[r, pl.ds(d * 16, 16)] = accs[d]

    issue(0, rows0, sem0)
    bufs = ((rows0, sem0), (rows1, sem1))

    def outer(o, carry):
        for b in range(2):
            r = o * 2 + b
            buf, sem = bufs[b]
            nbuf, nsem = bufs[1 - b]

            @pl.when(r + 1 < ROWS_PER_W)
            def _():
                issue(r + 1, nbuf, nsem)

            drain(buf, sem)
            reduce_row(buf, r)
        return carry

    lax.fori_loop(0, ROWS_PER_W // 2, outer, 0)
    pltpu.sync_copy(out_v, out_hbm.at[pl.ds(base, ROWS_PER_W), :])


@functools.lru_cache(maxsize=None)
def _make_bag_sum():
  return pl.kernel(
    _bag_body,
    out_type=jax.ShapeDtypeStruct((BATCH, D), jnp.float32),
    mesh=plsc.VectorSubcoreMesh(core_axis_name="c", subcore_axis_name="s",
                                num_cores=NC, num_subcores=NS),
    scratch_types=[
        pltpu.VMEM((ROWS_PER_W, SEQ), jnp.int32),
        pltpu.VMEM((SEQ, D), jnp.float32),
        pltpu.VMEM((SEQ, D), jnp.float32),
        pltpu.VMEM((ROWS_PER_W, D), jnp.float32),
        pltpu.SemaphoreType.DMA,
        pltpu.SemaphoreType.DMA,
    ],
    compiler_params=pltpu.CompilerParams(use_tc_tiling_on_sc=False),
  )


BT = 512


def _fc_body(emb_ref, w_ref, b_ref, out_ref):
    emb = emb_ref[...]
    norm = jnp.sqrt(jnp.sum(emb * emb, axis=1, keepdims=True))
    embn = emb / jnp.maximum(norm, 1e-12)
    out = lax.dot_general(embn, w_ref[...], (((1,), (1,)), ((), ())),
                          preferred_element_type=jnp.float32)
    out_ref[...] = out + b_ref[...]


def _fc(sums, W, b2d):
    return pl.pallas_call(
        _fc_body,
        grid=(BATCH // BT,),
        in_specs=[
            pl.BlockSpec((BT, D), lambda i: (i, 0)),
            pl.BlockSpec((OUT_DIM, D), lambda i: (0, 0)),
            pl.BlockSpec((1, OUT_DIM), lambda i: (0, 0)),
        ],
        out_specs=pl.BlockSpec((BT, OUT_DIM), lambda i: (i, 0)),
        out_shape=jax.ShapeDtypeStruct((BATCH, OUT_DIM), jnp.float32),
    )(sums, W, b2d)


def kernel(name_idxs, name_len, desc_idxs, desc_len, union_idxs, union_len, table, W, b):
    idx = union_idxs.astype(jnp.int32)
    idx = jnp.where(idx < SPLIT, 2 * idx, 2 * (idx - SPLIT) + 1)
    table_lin = _relayout(table.T).reshape(2 * SPLIT, D)
    sums = _make_bag_sum()(idx, table_lin)
    return _fc(sums, W, b.reshape(1, OUT_DIM))


# speedup vs baseline: 5.2112x; 1.1218x over previous
import functools

import jax
import jax.numpy as jnp
from jax import lax
from jax.experimental import pallas as pl
from jax.experimental.pallas import tpu as pltpu
from jax.experimental.pallas import tpu_sc as plsc

BATCH = 4096
SEQ = 200
D = 64
OUT_DIM = 1000
VOCAB_ROWS = 1000000

NC, NS = 2, 16
NW = NC * NS
ROWS_PER_W = BATCH // NW
CHUNK = 40
NCHUNK = SEQ // CHUNK

TC_COLS = 8960
N_RELAYOUT = 56
SPLIT = TC_COLS * N_RELAYOUT


def _relayout_body(lo_ref, hi_ref, out_ref):
    stacked = jnp.concatenate([lo_ref[...], hi_ref[...]], axis=0)
    row = jax.lax.broadcasted_iota(jnp.int32, (2 * D, 2 * D), 0)
    col = jax.lax.broadcasted_iota(jnp.int32, (2 * D, 2 * D), 1)
    ident = (row == col).astype(jnp.float32)
    out_ref[...] = jax.lax.dot_general(
        stacked, ident, (((0,), (0,)), ((), ())),
        preferred_element_type=jnp.float32)


def _relayout(tableT):
    return pl.pallas_call(
        _relayout_body,
        grid=(N_RELAYOUT,),
        in_specs=[
            pl.BlockSpec((D, TC_COLS), lambda i: (0, i)),
            pl.BlockSpec((D, TC_COLS), lambda i: (0, i + N_RELAYOUT)),
        ],
        out_specs=pl.BlockSpec((TC_COLS, 2 * D), lambda i: (i, 0)),
        out_shape=jax.ShapeDtypeStruct((SPLIT, 2 * D), jnp.float32),
    )(tableT, tableT)


def _bag_body(idx_hbm, table_hbm, out_hbm, idx_v, rows0, rows1, out_v, sem0, sem1):
    wid = lax.axis_index("s") * NC + lax.axis_index("c")
    base = wid * ROWS_PER_W

    pltpu.sync_copy(idx_hbm.at[pl.ds(base, ROWS_PER_W), :], idx_v)

    def issue(r, buf, sem):
        for j in range(NCHUNK):
            pltpu.async_copy(
                table_hbm.at[idx_v.at[r, pl.ds(j * CHUNK, CHUNK)]],
                buf.at[pl.ds(j * CHUNK, CHUNK), :],
                sem,
            )

    def drain(buf, sem):
        for j in range(NCHUNK):
            pltpu.make_async_copy(
                table_hbm.at[idx_v.at[0, pl.ds(0, CHUNK)]],
                buf.at[pl.ds(j * CHUNK, CHUNK), :],
                sem,
            ).wait()

    def reduce_row(buf, r):
        def body(i, accs):
            return tuple(a + buf[i, pl.ds(d * 16, 16)] for d, a in enumerate(accs))
        accs = lax.fori_loop(0, SEQ, body,
                             tuple(jnp.zeros((16,), jnp.float32) for _ in range(4)),
                             unroll=8)
        for d in range(4):
            out_v[r, pl.ds(d * 16, 16)] = accs[d]

    issue(0, rows0, sem0)
    bufs = ((rows0, sem0), (rows1, sem1))

    def outer(o, carry):
        for b in range(2):
            r = o * 2 + b
            buf, sem = bufs[b]
            nbuf, nsem = bufs[1 - b]

            @pl.when(r + 1 < ROWS_PER_W)
            def _():
                issue(r + 1, nbuf, nsem)

            drain(buf, sem)
            reduce_row(buf, r)
        return carry

    lax.fori_loop(0, ROWS_PER_W // 2, outer, 0)
    pltpu.sync_copy(out_v, out_hbm.at[pl.ds(base, ROWS_PER_W), :])


@functools.lru_cache(maxsize=None)
def _make_bag_sum():
  return pl.kernel(
    _bag_body,
    out_type=jax.ShapeDtypeStruct((BATCH, D), jnp.float32),
    mesh=plsc.VectorSubcoreMesh(core_axis_name="c", subcore_axis_name="s",
                                num_cores=NC, num_subcores=NS),
    scratch_types=[
        pltpu.VMEM((ROWS_PER_W, SEQ), jnp.int32),
        pltpu.VMEM((SEQ, D), jnp.float32),
        pltpu.VMEM((SEQ, D), jnp.float32),
        pltpu.VMEM((ROWS_PER_W, D), jnp.float32),
        pltpu.SemaphoreType.DMA,
        pltpu.SemaphoreType.DMA,
    ],
    compiler_params=pltpu.CompilerParams(use_tc_tiling_on_sc=False),
  )


BT = 512


def _fc_body(emb_ref, w_ref, b_ref, out_ref):
    emb = emb_ref[...]
    norm = jnp.sqrt(jnp.sum(emb * emb, axis=1, keepdims=True))
    embn = emb / jnp.maximum(norm, 1e-12)
    out = lax.dot_general(w_ref[...], embn, (((1,), (1,)), ((), ())),
                          preferred_element_type=jnp.float32)
    out_ref[...] = out + b_ref[...]


def _fc(sums, W, b2d):
    return pl.pallas_call(
        _fc_body,
        grid=(BATCH // BT,),
        in_specs=[
            pl.BlockSpec((BT, D), lambda i: (i, 0)),
            pl.BlockSpec((OUT_DIM, D), lambda i: (0, 0)),
            pl.BlockSpec((OUT_DIM, 1), lambda i: (0, 0)),
        ],
        out_specs=pl.BlockSpec((OUT_DIM, BT), lambda i: (0, i)),
        out_shape=jax.ShapeDtypeStruct((OUT_DIM, BATCH), jnp.float32),
    )(sums, W, b2d)


def kernel(name_idxs, name_len, desc_idxs, desc_len, union_idxs, union_len, table, W, b):
    idx = union_idxs.astype(jnp.int32)
    idx = jnp.where(idx < SPLIT, 2 * idx, 2 * (idx - SPLIT) + 1)
    table_lin = _relayout(table.T).reshape(2 * SPLIT, D)
    sums = _make_bag_sum()(idx, table_lin)
    return _fc(sums, W, b.reshape(OUT_DIM, 1)).T


# speedup vs baseline: 5.2376x; 1.0051x over previous
import functools

import jax
import jax.numpy as jnp
from jax import lax
from jax.experimental import pallas as pl
from jax.experimental.pallas import tpu as pltpu
from jax.experimental.pallas import tpu_sc as plsc

BATCH = 4096
SEQ = 200
D = 64
OUT_DIM = 1000
VOCAB_ROWS = 1000000

NC, NS = 2, 16
NW = NC * NS
ROWS_PER_W = BATCH // NW
CHUNK = 40
NCHUNK = SEQ // CHUNK

TC_COLS = 17920
N_RELAYOUT = 28
SPLIT = TC_COLS * N_RELAYOUT


def _relayout_body(lo_ref, hi_ref, out_ref):
    stacked = jnp.concatenate([lo_ref[...], hi_ref[...]], axis=0)
    row = jax.lax.broadcasted_iota(jnp.int32, (2 * D, 2 * D), 0)
    col = jax.lax.broadcasted_iota(jnp.int32, (2 * D, 2 * D), 1)
    ident = (row == col).astype(jnp.float32)
    out_ref[...] = jax.lax.dot_general(
        stacked, ident, (((0,), (0,)), ((), ())),
        preferred_element_type=jnp.float32)


def _relayout(tableT):
    return pl.pallas_call(
        _relayout_body,
        grid=(N_RELAYOUT,),
        in_specs=[
            pl.BlockSpec((D, TC_COLS), lambda i: (0, i)),
            pl.BlockSpec((D, TC_COLS), lambda i: (0, i + N_RELAYOUT)),
        ],
        out_specs=pl.BlockSpec((TC_COLS, 2 * D), lambda i: (i, 0)),
        out_shape=jax.ShapeDtypeStruct((SPLIT, 2 * D), jnp.float32),
    )(tableT, tableT)


def _bag_body(idx_hbm, table_hbm, out_hbm, idx_v, rows0, rows1, out_v, sem0, sem1):
    wid = lax.axis_index("s") * NC + lax.axis_index("c")
    base = wid * ROWS_PER_W

    pltpu.sync_copy(idx_hbm.at[pl.ds(base, ROWS_PER_W), :], idx_v)

    def issue(r, buf, sem):
        for j in range(NCHUNK):
            pltpu.async_copy(
                table_hbm.at[idx_v.at[r, pl.ds(j * CHUNK, CHUNK)]],
                buf.at[pl.ds(j * CHUNK, CHUNK), :],
                sem,
            )

    def drain(buf, sem):
        for j in range(NCHUNK):
            pltpu.make_async_copy(
                table_hbm.at[idx_v.at[0, pl.ds(0, CHUNK)]],
                buf.at[pl.ds(j * CHUNK, CHUNK), :],
                sem,
            ).wait()

    def reduce_row(buf, r):
        def body(i, accs):
            return tuple(a + buf[i, pl.ds(d * 16, 16)] for d, a in enumerate(accs))
        accs = lax.fori_loop(0, SEQ, body,
                             tuple(jnp.zeros((16,), jnp.float32) for _ in range(4)),
                             unroll=8)
        for d in range(4):
            out_v[r, pl.ds(d * 16, 16)] = accs[d]

    issue(0, rows0, sem0)
    bufs = ((rows0, sem0), (rows1, sem1))

    def outer(o, carry):
        for b in range(2):
            r = o * 2 + b
            buf, sem = bufs[b]
            nbuf, nsem = bufs[1 - b]

            @pl.when(r + 1 < ROWS_PER_W)
            def _():
                issue(r + 1, nbuf, nsem)

            drain(buf, sem)
            reduce_row(buf, r)
        return carry

    lax.fori_loop(0, ROWS_PER_W // 2, outer, 0)
    pltpu.sync_copy(out_v, out_hbm.at[pl.ds(base, ROWS_PER_W), :])


@functools.lru_cache(maxsize=None)
def _make_bag_sum():
  return pl.kernel(
    _bag_body,
    out_type=jax.ShapeDtypeStruct((BATCH, D), jnp.float32),
    mesh=plsc.VectorSubcoreMesh(core_axis_name="c", subcore_axis_name="s",
                                num_cores=NC, num_subcores=NS),
    scratch_types=[
        pltpu.VMEM((ROWS_PER_W, SEQ), jnp.int32),
        pltpu.VMEM((SEQ, D), jnp.float32),
        pltpu.VMEM((SEQ, D), jnp.float32),
        pltpu.VMEM((ROWS_PER_W, D), jnp.float32),
        pltpu.SemaphoreType.DMA,
        pltpu.SemaphoreType.DMA,
    ],
    compiler_params=pltpu.CompilerParams(use_tc_tiling_on_sc=False),
  )


BT = 512


def _fc_body(emb_ref, w_ref, b_ref, out_ref):
    emb = emb_ref[...]
    norm = jnp.sqrt(jnp.sum(emb * emb, axis=1, keepdims=True))
    embn = emb / jnp.maximum(norm, 1e-12)
    out = lax.dot_general(w_ref[...], embn, (((1,), (1,)), ((), ())),
                          preferred_element_type=jnp.float32)
    out_ref[...] = out + b_ref[...]


def _fc(sums, W, b2d):
    return pl.pallas_call(
        _fc_body,
        grid=(BATCH // BT,),
        in_specs=[
            pl.BlockSpec((BT, D), lambda i: (i, 0)),
            pl.BlockSpec((OUT_DIM, D), lambda i: (0, 0)),
            pl.BlockSpec((OUT_DIM, 1), lambda i: (0, 0)),
        ],
        out_specs=pl.BlockSpec((OUT_DIM, BT), lambda i: (0, i)),
        out_shape=jax.ShapeDtypeStruct((OUT_DIM, BATCH), jnp.float32),
    )(sums, W, b2d)


def kernel(name_idxs, name_len, desc_idxs, desc_len, union_idxs, union_len, table, W, b):
    idx = union_idxs.astype(jnp.int32)
    idx = jnp.where(idx < SPLIT, 2 * idx, 2 * (idx - SPLIT) + 1)
    table_lin = _relayout(table.T).reshape(2 * SPLIT, D)
    sums = _make_bag_sum()(idx, table_lin)
    return _fc(sums, W, b.reshape(OUT_DIM, 1)).T
